# baseline (speedup 1.0000x reference)
import functools

import jax
import jax.numpy as jnp
from jax import lax
from jax.experimental import pallas as pl
from jax.experimental.pallas import tpu as pltpu
from jax.experimental.pallas import tpu_sc as plsc

_EPS = 1e-12
_LANES = 16
_NC = 2
_NS = 16
_NW = _NC * _NS


def _hsum(v, idxs):
    dnums = lax.GatherDimensionNumbers(
        offset_dims=(), collapsed_slice_dims=(0,), start_index_map=(0,))
    for idx in idxs:
        v = v + lax.gather(v, idx[:, None], dnums, slice_sizes=(1,),
                           mode=lax.GatherScatterMode.PROMISE_IN_BOUNDS)
    return v


def _rsqrt_vec(d):
    i = plsc.bitcast(d, jnp.int32)
    y = plsc.bitcast(jnp.int32(0x5F3759DF) - lax.shift_right_logical(i, 1),
                     jnp.float32)
    for _ in range(3):
        y = y * (1.5 - 0.5 * d * y * y)
    return y


def _build_sc_kernel(B, L, H, CH):
    N = B * L
    rows_per_w = N // _NW
    chunks = rows_per_w // CH
    nvec = H // _LANES
    mesh = plsc.VectorSubcoreMesh(core_axis_name="c", subcore_axis_name="s")

    @functools.partial(
        pl.kernel,
        out_type=jax.ShapeDtypeStruct((B, L, H), jnp.float32),
        mesh=mesh,
        compiler_params=pltpu.CompilerParams(needs_layout_passes=False),
        scratch_types=[
            pltpu.VMEM((rows_per_w, 3), jnp.int32),
            pltpu.VMEM((rows_per_w,), jnp.int32),
            pltpu.VMEM((rows_per_w,), jnp.int32),
            pltpu.VMEM((rows_per_w,), jnp.int32),
            pltpu.VMEM((rows_per_w,), jnp.int32),
            [pltpu.VMEM((CH, H), jnp.float32) for _ in range(2)],
            [pltpu.VMEM((CH, H), jnp.float32) for _ in range(2)],
            [pltpu.VMEM((CH, H), jnp.float32) for _ in range(2)],
            [pltpu.VMEM((CH, H), jnp.float32) for _ in range(2)],
            [pltpu.VMEM((CH, H), jnp.float32) for _ in range(2)],
            [pltpu.VMEM((CH, H), jnp.float32) for _ in range(2)],
            pltpu.VMEM((H,), jnp.float32),
            pltpu.VMEM((H,), jnp.float32),
            pltpu.VMEM((H,), jnp.float32),
            pltpu.SemaphoreType.DMA,
            [pltpu.SemaphoreType.DMA for _ in range(2)],
            [pltpu.SemaphoreType.DMA for _ in range(2)],
        ],
    )
    def emb(ids_hbm, tsv_hbm, wword_hbm, wpos_hbm, wtype_hbm, wa_hbm,
            wb_hbm, wc_hbm, gamma_hbm, beta_hbm,
            out_hbm,
            tsv_buf, idx_w, idx_a, idx_b, idx_c,
            buf_w, buf_a, buf_b, buf_c, buf_p, out_buf,
            type_row, gamma_row, beta_row,
            sem_i, sem_g, sem_o):
        wid = lax.axis_index("s") * _NC + lax.axis_index("c")
        wbase = wid * rows_per_w
        b = wbase // L
        l0w = wbase - b * L

        span = pl.ds(l0w, rows_per_w)
        icps = [
            pltpu.async_copy(ids_hbm.at[b, span], idx_w, sem_i),
            pltpu.async_copy(tsv_hbm.at[b, span, :], tsv_buf, sem_i),
            pltpu.async_copy(wtype_hbm.at[0], type_row, sem_i),
            pltpu.async_copy(gamma_hbm, gamma_row, sem_i),
            pltpu.async_copy(beta_hbm, beta_row, sem_i),
        ]
        for cp in icps:
            cp.wait()

        lane = lax.iota(jnp.int32, _LANES)
        bfly = [lane ^ k for k in (8, 4, 2, 1)]

        zero = jnp.zeros((_LANES,), jnp.int32)

        def split_body(g, carry):
            rows = g * _LANES + lane
            sl = pl.ds(g * _LANES, _LANES)
            idx_a[sl] = plsc.load_gather(tsv_buf, [rows, zero])
            idx_b[sl] = plsc.load_gather(tsv_buf, [rows, zero + 1])
            idx_c[sl] = plsc.load_gather(tsv_buf, [rows, zero + 2])
            return carry

        lax.fori_loop(0, rows_per_w // _LANES, split_body, 0)

        t_vs = [type_row[pl.ds(j * _LANES, _LANES)] for j in range(nvec)]
        g_vs = [gamma_row[pl.ds(j * _LANES, _LANES)] for j in range(nvec)]
        b_vs = [beta_row[pl.ds(j * _LANES, _LANES)] for j in range(nvec)]

        g_cp = [None, None]
        o_cp = [None, None]

        def fire_gathers(c):
            s = c & 1
            sl = pl.ds(c * CH, CH)
            l0 = lax.rem(wbase + c * CH, L)
            g_cp[s] = [
                pltpu.async_copy(wword_hbm.at[idx_w.at[sl]], buf_w[s],
                                 sem_g[s]),
                pltpu.async_copy(wa_hbm.at[idx_a.at[sl]], buf_a[s],
                                 sem_g[s]),
                pltpu.async_copy(wb_hbm.at[idx_b.at[sl]], buf_b[s],
                                 sem_g[s]),
                pltpu.async_copy(wc_hbm.at[idx_c.at[sl]], buf_c[s],
                                 sem_g[s]),
                pltpu.async_copy(wpos_hbm.at[pl.ds(l0, CH)], buf_p[s],
                                 sem_g[s]),
            ]

        inv_h = jnp.float32(1.0 / H)
        eps = jnp.float32(_EPS)

        fire_gathers(0)
        for c in range(chunks):
            s = c & 1
            if c + 1 < chunks:
                fire_gathers(c + 1)
            for cp in g_cp[s]:
                cp.wait()
            if o_cp[s] is not None:
                o_cp[s].wait()

            bw, ba, bb, bc, bp, bo = (buf_w[s], buf_a[s], buf_b[s],
                                      buf_c[s], buf_p[s], out_buf[s])

            def token_body(t2, carry, bw=bw, ba=ba, bb=bb, bc=bc, bp=bp,
                          bo=bo):
                for u in range(2):
                    t = t2 * 2 + u
                    vs = []
                    ssum = None
                    sq = None
                    for j in range(nvec):
                        sl = pl.ds(j * _LANES, _LANES)
                        v = (bw[t, sl] + ba[t, sl] + bb[t, sl]
                             + bc[t, sl] + bp[t, sl] + t_vs[j])
                        vs.append(v)
                        ssum = v if ssum is None else ssum + v
                        sq = v * v if sq is None else sq + v * v
                    mean = _hsum(ssum, bfly) * inv_h
                    var = _hsum(sq, bfly) * inv_h - mean * mean
                    rstd = _rsqrt_vec(var + eps)
                    for j in range(nvec):
                        sl = pl.ds(j * _LANES, _LANES)
                        bo[t, sl] = (vs[j] - mean) * rstd * g_vs[j] + b_vs[j]
                return carry

            lax.fori_loop(0, CH // 2, token_body, 0)
            lc = lax.rem(wbase + c * CH, L)
            o_cp[s] = pltpu.async_copy(
                bo, out_hbm.at[b, pl.ds(lc, CH), :], sem_o[s])

        o_cp[0].wait()
        o_cp[1].wait()

    return emb


def kernel(input_ids, tok_struct_vec, sent_struct_vec, W_word, W_pos,
           W_type, W_a, W_b, W_c, gamma, beta):
    B, L = input_ids.shape
    H = W_word.shape[1]
    N = B * L
    assert N % _NW == 0
    CH = 32
    rows_per_w = N // _NW
    assert rows_per_w % CH == 0 and L % rows_per_w == 0

    emb = _build_sc_kernel(B, L, H, CH)
    return emb(input_ids.astype(jnp.int32), tok_struct_vec.astype(jnp.int32),
               W_word, W_pos, W_type, W_a, W_b, W_c, gamma, beta)


# speedup vs baseline: 1.1803x; 1.1803x over previous
import functools

import jax
import jax.numpy as jnp
from jax import lax
from jax.experimental import pallas as pl
from jax.experimental.pallas import tpu as pltpu
from jax.experimental.pallas import tpu_sc as plsc

_EPS = 1e-12
_LANES = 16
_NC = 2
_NS = 16
_NW = _NC * _NS


def _hsum(v, idxs):
    dnums = lax.GatherDimensionNumbers(
        offset_dims=(), collapsed_slice_dims=(0,), start_index_map=(0,))
    for idx in idxs:
        v = v + lax.gather(v, idx[:, None], dnums, slice_sizes=(1,),
                           mode=lax.GatherScatterMode.PROMISE_IN_BOUNDS)
    return v


def _rsqrt_vec(d):
    i = plsc.bitcast(d, jnp.int32)
    y = plsc.bitcast(jnp.int32(0x5F3759DF) - lax.shift_right_logical(i, 1),
                     jnp.float32)
    for _ in range(3):
        y = y * (1.5 - 0.5 * d * y * y)
    return y


def _build_sc_kernel(N, H, L, CH):
    rows_per_w = N // _NW
    chunks = rows_per_w // CH
    nvec = H // _LANES
    mesh = plsc.VectorSubcoreMesh(core_axis_name="c", subcore_axis_name="s")

    @functools.partial(
        pl.kernel,
        out_type=jax.ShapeDtypeStruct((N, H), jnp.float32),
        mesh=mesh,
        compiler_params=pltpu.CompilerParams(needs_layout_passes=False),
        scratch_types=[
            pltpu.VMEM((rows_per_w,), jnp.int32),
            pltpu.VMEM((rows_per_w,), jnp.int32),
            pltpu.VMEM((rows_per_w,), jnp.int32),
            pltpu.VMEM((rows_per_w,), jnp.int32),
            [pltpu.VMEM((CH, H), jnp.float32) for _ in range(2)],
            [pltpu.VMEM((CH, H), jnp.float32) for _ in range(2)],
            [pltpu.VMEM((CH, H), jnp.float32) for _ in range(2)],
            [pltpu.VMEM((CH, H), jnp.float32) for _ in range(2)],
            [pltpu.VMEM((CH, H), jnp.float32) for _ in range(2)],
            [pltpu.VMEM((CH, H), jnp.float32) for _ in range(2)],
            pltpu.VMEM((H,), jnp.float32),
            pltpu.VMEM((H,), jnp.float32),
            pltpu.VMEM((H,), jnp.float32),
            pltpu.SemaphoreType.DMA,
            [pltpu.SemaphoreType.DMA for _ in range(2)],
            [pltpu.SemaphoreType.DMA for _ in range(2)],
        ],
    )
    def emb(ids_hbm, para_hbm, sent_hbm, tok_hbm, wword_hbm, wpos_hbm,
            wtype_hbm, wa_hbm, wb_hbm, wc_hbm, gamma_hbm, beta_hbm,
            out_hbm,
            idx_w, idx_a, idx_b, idx_c,
            buf_w, buf_a, buf_b, buf_c, buf_p, out_buf,
            type_row, gamma_row, beta_row,
            sem_i, sem_g, sem_o):
        wid = lax.axis_index("s") * _NC + lax.axis_index("c")
        wbase = wid * rows_per_w

        span = pl.ds(wbase, rows_per_w)
        icps = [pltpu.async_copy(ids_hbm.at[span], idx_w, sem_i),
                pltpu.async_copy(para_hbm.at[span], idx_a, sem_i),
                pltpu.async_copy(sent_hbm.at[span], idx_b, sem_i),
                pltpu.async_copy(tok_hbm.at[span], idx_c, sem_i),
                pltpu.async_copy(wtype_hbm.at[0], type_row, sem_i),
                pltpu.async_copy(gamma_hbm, gamma_row, sem_i),
                pltpu.async_copy(beta_hbm, beta_row, sem_i)]
        for cp in icps:
            cp.wait()

        lane = lax.iota(jnp.int32, _LANES)
        bfly = [lane ^ k for k in (8, 4, 2, 1)]
        t_vs = [type_row[pl.ds(j * _LANES, _LANES)] for j in range(nvec)]
        g_vs = [gamma_row[pl.ds(j * _LANES, _LANES)] for j in range(nvec)]
        b_vs = [beta_row[pl.ds(j * _LANES, _LANES)] for j in range(nvec)]

        g_cp = [None, None]
        o_cp = [None, None]

        def fire_gathers(c):
            s = c & 1
            sl = pl.ds(c * CH, CH)
            l0 = lax.rem(wbase + c * CH, L)
            g_cp[s] = [
                pltpu.async_copy(wword_hbm.at[idx_w.at[sl]], buf_w[s],
                                 sem_g[s]),
                pltpu.async_copy(wa_hbm.at[idx_a.at[sl]], buf_a[s],
                                 sem_g[s]),
                pltpu.async_copy(wb_hbm.at[idx_b.at[sl]], buf_b[s],
                                 sem_g[s]),
                pltpu.async_copy(wc_hbm.at[idx_c.at[sl]], buf_c[s],
                                 sem_g[s]),
                pltpu.async_copy(wpos_hbm.at[pl.ds(l0, CH)], buf_p[s],
                                 sem_g[s]),
            ]

        inv_h = jnp.float32(1.0 / H)
        eps = jnp.float32(_EPS)

        fire_gathers(0)
        for c in range(chunks):
            s = c & 1
            if c + 1 < chunks:
                fire_gathers(c + 1)
            for cp in g_cp[s]:
                cp.wait()
            if o_cp[s] is not None:
                o_cp[s].wait()

            bw, ba, bb, bc, bp, bo = (buf_w[s], buf_a[s], buf_b[s],
                                      buf_c[s], buf_p[s], out_buf[s])

            def token_body(t2, carry, bw=bw, ba=ba, bb=bb, bc=bc, bp=bp,
                          bo=bo):
                for u in range(2):
                    t = t2 * 2 + u
                    vs = []
                    ssum = None
                    sq = None
                    for j in range(nvec):
                        sl = pl.ds(j * _LANES, _LANES)
                        v = (bw[t, sl] + ba[t, sl] + bb[t, sl]
                             + bc[t, sl] + bp[t, sl] + t_vs[j])
                        vs.append(v)
                        ssum = v if ssum is None else ssum + v
                        sq = v * v if sq is None else sq + v * v
                    mean = _hsum(ssum, bfly) * inv_h
                    var = _hsum(sq, bfly) * inv_h - mean * mean
                    rstd = _rsqrt_vec(var + eps)
                    for j in range(nvec):
                        sl = pl.ds(j * _LANES, _LANES)
                        bo[t, sl] = (vs[j] - mean) * rstd * g_vs[j] + b_vs[j]
                return carry

            lax.fori_loop(0, CH // 2, token_body, 0)
            o_cp[s] = pltpu.async_copy(
                bo, out_hbm.at[pl.ds(wbase + c * CH, CH)], sem_o[s])

        o_cp[0].wait()
        o_cp[1].wait()

    return emb


def kernel(input_ids, tok_struct_vec, sent_struct_vec, W_word, W_pos,
           W_type, W_a, W_b, W_c, gamma, beta):
    B, L = input_ids.shape
    H = W_word.shape[1]
    N = B * L
    assert N % _NW == 0
    CH = 64
    assert (N // _NW) % CH == 0 and L % CH == 0

    ids = input_ids.reshape(N).astype(jnp.int32)
    para = tok_struct_vec[:, :, 0].reshape(N).astype(jnp.int32)
    sent = tok_struct_vec[:, :, 1].reshape(N).astype(jnp.int32)
    tok = tok_struct_vec[:, :, 2].reshape(N).astype(jnp.int32)

    emb = _build_sc_kernel(N, H, L, CH)
    out = emb(ids, para, sent, tok, W_word, W_pos, W_type, W_a, W_b, W_c,
              gamma, beta)
    return out.reshape(B, L, H)
